# Initial kernel scaffold; baseline (speedup 1.0000x reference)
import jax
import jax.numpy as jnp
from jax.experimental import pallas as pl


def kernel(x, W, b, W1, b1, W2, b2, A, Bm):
    raise NotImplementedError("write your pallas kernel here")


# speedup vs baseline: 10.0165x; 10.0165x over previous
import functools

import jax
import jax.numpy as jnp
from jax.experimental import pallas as pl
from jax.experimental.pallas import tpu as pltpu

_K = 2
_SCALING = 16.0 / 8.0


def _router_body(x_ref, w1_ref, b1_ref, w2_ref, b2_ref,
                 probs_ref, tw_ref, ti_ref, acc_ref, *, t_total, n_exp):
    t = pl.program_id(0)

    @pl.when(t == 0)
    def _init():
        acc_ref[...] = jnp.zeros_like(acc_ref)

    acc_ref[...] += jnp.sum(x_ref[...], axis=1)

    @pl.when(t == pl.num_programs(0) - 1)
    def _finish():
        pooled = acc_ref[...] / t_total
        h = jax.lax.dot_general(
            pooled, w1_ref[...], (((1,), (1,)), ((), ())),
            preferred_element_type=jnp.float32) + b1_ref[...]
        h = jnp.maximum(h, 0.0)
        logits = jax.lax.dot_general(
            h, w2_ref[...], (((1,), (1,)), ((), ())),
            preferred_element_type=jnp.float32) + b2_ref[...]
        m = jnp.max(logits, axis=-1, keepdims=True)
        ex = jnp.exp(logits - m)
        probs = ex / jnp.sum(ex, axis=-1, keepdims=True)
        probs_ref[...] = probs

        iota = jax.lax.broadcasted_iota(jnp.int32, probs.shape, 1)
        m1 = jnp.max(probs, axis=-1, keepdims=True)
        i1 = jnp.min(jnp.where(probs == m1, iota, n_exp), axis=-1,
                     keepdims=True)
        masked = jnp.where(iota == i1, -1.0, probs)
        m2 = jnp.max(masked, axis=-1, keepdims=True)
        i2 = jnp.min(jnp.where(masked == m2, iota, n_exp), axis=-1,
                     keepdims=True)
        denom = jnp.maximum(m1 + m2, 1e-6)
        tw_ref[...] = jnp.concatenate([m1, m2], axis=-1) / denom
        ti_ref[...] = jnp.concatenate([i1, i2], axis=-1)


def _main_body(x_ref, w_ref, bias_ref, asel_ref, bsel_ref, scale_ref, o_ref):
    x = x_ref[0]
    acc = jax.lax.dot_general(
        x, w_ref[...], (((1,), (1,)), ((), ())),
        preferred_element_type=jnp.float32) + bias_ref[...]
    t1 = jax.lax.dot_general(
        x, asel_ref[0], (((1,), (1,)), ((), ())),
        preferred_element_type=jnp.float32)
    delta = jax.lax.dot_general(
        t1 * scale_ref[0], bsel_ref[0], (((1,), (0,)), ((), ())),
        preferred_element_type=jnp.float32)
    o_ref[0] = acc + delta


def kernel(x, W, b, W1, b1, W2, b2, A, Bm):
    B, T, D = x.shape
    OUT = W.shape[0]
    HID = W1.shape[0]
    E = W2.shape[0]
    R = A.shape[1]
    K = _K
    KR = K * R

    RTT = 512
    router = pl.pallas_call(
        functools.partial(_router_body, t_total=float(T), n_exp=E),
        grid=(T // RTT,),
        in_specs=[
            pl.BlockSpec((B, RTT, D), lambda t: (0, t, 0)),
            pl.BlockSpec((HID, D), lambda t: (0, 0)),
            pl.BlockSpec((1, HID), lambda t: (0, 0)),
            pl.BlockSpec((E, HID), lambda t: (0, 0)),
            pl.BlockSpec((1, E), lambda t: (0, 0)),
        ],
        out_specs=[
            pl.BlockSpec((B, E), lambda t: (0, 0)),
            pl.BlockSpec((B, K), lambda t: (0, 0)),
            pl.BlockSpec((B, K), lambda t: (0, 0)),
        ],
        out_shape=[
            jax.ShapeDtypeStruct((B, E), jnp.float32),
            jax.ShapeDtypeStruct((B, K), jnp.float32),
            jax.ShapeDtypeStruct((B, K), jnp.int32),
        ],
        scratch_shapes=[pltpu.VMEM((B, D), jnp.float32)],
    )
    probs, tw, ti = router(x, W1, b1.reshape(1, HID), W2, b2.reshape(1, E))

    A_sel = A[ti].reshape(B, KR, D)
    Bmt = Bm.transpose(0, 2, 1)
    B_sel = Bmt[ti].reshape(B, KR, OUT)
    scale = jnp.repeat(tw * _SCALING, R, axis=1).reshape(B, 1, KR)

    TT = 512
    main = pl.pallas_call(
        _main_body,
        grid=(B, T // TT),
        in_specs=[
            pl.BlockSpec((1, TT, D), lambda bb, t: (bb, t, 0)),
            pl.BlockSpec((OUT, D), lambda bb, t: (0, 0)),
            pl.BlockSpec((1, OUT), lambda bb, t: (0, 0)),
            pl.BlockSpec((1, KR, D), lambda bb, t: (bb, 0, 0)),
            pl.BlockSpec((1, KR, OUT), lambda bb, t: (bb, 0, 0)),
            pl.BlockSpec((1, 1, KR), lambda bb, t: (bb, 0, 0)),
        ],
        out_specs=pl.BlockSpec((1, TT, OUT), lambda bb, t: (bb, t, 0)),
        out_shape=jax.ShapeDtypeStruct((B, T, OUT), jnp.float32),
    )
    final_out = main(x, W, b.reshape(1, OUT), A_sel, B_sel, scale)
    return (final_out, probs)
